# baseline (speedup 1.0000x reference)
import jax
import jax.numpy as jnp
from jax.experimental import pallas as pl

_DIM = 2048
_N_EXPERTS = 64
_TOPK = 8
_ROWS = 8192
_BLOCK_R = 2048


def _gate_block(x_ref, w_ref, wts_ref, idx_ref):
    st = jax.lax.dot_general(
        w_ref[...], x_ref[...],
        (((1,), (1,)), ((), ())),
        preferred_element_type=jnp.float32,
    )
    m = jnp.max(st, axis=0, keepdims=True)
    e = jnp.exp(st - m)
    recip = 1.0 / jnp.sum(e, axis=0, keepdims=True)

    iota = jax.lax.broadcasted_iota(jnp.int32, st.shape, 0).astype(jnp.float32)
    wts_rows = []
    idx_rows = []
    work = st
    neg = jnp.float32(-jnp.inf)
    for k in range(_TOPK):
        mx = m if k == 0 else jnp.max(work, axis=0, keepdims=True)
        ix = jnp.min(jnp.where(work == mx, iota, jnp.float32(_N_EXPERTS)),
                     axis=0, keepdims=True)
        wts_rows.append(jnp.exp(mx - m) * recip)
        idx_rows.append(ix)
        work = jnp.where(iota == ix, neg, work)
    wts_ref[...] = jnp.concatenate(wts_rows, axis=0)
    idx_ref[...] = jnp.concatenate(idx_rows, axis=0).astype(jnp.int32)


def kernel(x, weight):
    grid = (_ROWS // _BLOCK_R,)
    wts_t, idx_t = pl.pallas_call(
        _gate_block,
        grid=grid,
        in_specs=[
            pl.BlockSpec((_BLOCK_R, _DIM), lambda i: (i, 0)),
            pl.BlockSpec((_N_EXPERTS, _DIM), lambda i: (0, 0)),
        ],
        out_specs=[
            pl.BlockSpec((_TOPK, _BLOCK_R), lambda i: (0, i)),
            pl.BlockSpec((_TOPK, _BLOCK_R), lambda i: (0, i)),
        ],
        out_shape=[
            jax.ShapeDtypeStruct((_TOPK, _ROWS), jnp.float32),
            jax.ShapeDtypeStruct((_TOPK, _ROWS), jnp.int32),
        ],
    )(x, weight)
    return wts_t.T, idx_t.T


# speedup vs baseline: 1.0074x; 1.0074x over previous
import jax
import jax.numpy as jnp
from jax.experimental import pallas as pl

_DIM = 2048
_N_EXPERTS = 64
_TOPK = 8
_ROWS = 8192
_BLOCK_R = 1024


def _gate_block(x_ref, w_ref, wts_ref, idx_ref):
    st = jax.lax.dot_general(
        w_ref[...], x_ref[...],
        (((1,), (1,)), ((), ())),
        preferred_element_type=jnp.float32,
    )
    m = jnp.max(st, axis=0, keepdims=True)
    e = jnp.exp(st - m)
    recip = 1.0 / jnp.sum(e, axis=0, keepdims=True)

    iota = jax.lax.broadcasted_iota(jnp.int32, st.shape, 0).astype(jnp.float32)
    wts_rows = []
    idx_rows = []
    work = st
    neg = jnp.float32(-jnp.inf)
    for k in range(_TOPK):
        mx = m if k == 0 else jnp.max(work, axis=0, keepdims=True)
        ix = jnp.min(jnp.where(work == mx, iota, jnp.float32(_N_EXPERTS)),
                     axis=0, keepdims=True)
        wts_rows.append(jnp.exp(mx - m) * recip)
        idx_rows.append(ix)
        work = jnp.where(iota == ix, neg, work)
    wts_ref[...] = jnp.concatenate(wts_rows, axis=0)
    idx_ref[...] = jnp.concatenate(idx_rows, axis=0).astype(jnp.int32)


def kernel(x, weight):
    grid = (_ROWS // _BLOCK_R,)
    wts_t, idx_t = pl.pallas_call(
        _gate_block,
        grid=grid,
        in_specs=[
            pl.BlockSpec((_BLOCK_R, _DIM), lambda i: (i, 0)),
            pl.BlockSpec((_N_EXPERTS, _DIM), lambda i: (0, 0)),
        ],
        out_specs=[
            pl.BlockSpec((_TOPK, _BLOCK_R), lambda i: (0, i)),
            pl.BlockSpec((_TOPK, _BLOCK_R), lambda i: (0, i)),
        ],
        out_shape=[
            jax.ShapeDtypeStruct((_TOPK, _ROWS), jnp.float32),
            jax.ShapeDtypeStruct((_TOPK, _ROWS), jnp.int32),
        ],
    )(x, weight)
    return wts_t.T, idx_t.T
